# baseline (speedup 1.0000x reference)
import functools

import jax
import jax.numpy as jnp
from jax import lax
from jax.experimental import pallas as pl
from jax.experimental.pallas import tpu as pltpu
from jax.experimental.pallas import tpu_sc as plsc

NNODES = 10000
NPAD = 10240
NC = 2
NS = 16
CHUNK = 128
NCHUNK = 160
IGRP = 20
NBUF = 5
EPAD = NS * NCHUNK * CHUNK
ROWS_PER_TILE = NPAD // NS
STAGE_PER_TILE = NNODES // NS


def _sc_degree():
  mesh = plsc.VectorSubcoreMesh(core_axis_name="c", subcore_axis_name="s",
                                num_cores=NC, num_subcores=NS)

  @functools.partial(
      pl.kernel,
      out_type=jax.ShapeDtypeStruct((NC, NPAD), jnp.float32),
      mesh=mesh,
      scratch_types=[
          pltpu.VMEM((NCHUNK, CHUNK), jnp.int32),
          pltpu.VMEM((CHUNK,), jnp.float32),
          pltpu.VMEM_SHARED((NPAD,), jnp.float32),
      ],
  )
  def deg_kernel(dstp_hbm, zeros_hbm, out_hbm, dst_v, ones_v, acc):
    cid = lax.axis_index("c")
    sid = lax.axis_index("s")
    for i in range(CHUNK // 16):
      ones_v[pl.ds(i * 16, 16)] = jnp.ones((16,), jnp.float32)
    pltpu.sync_copy(zeros_hbm, acc.at[pl.ds(sid * ROWS_PER_TILE, ROWS_PER_TILE)])
    plsc.subcore_barrier()
    pltpu.sync_copy(dstp_hbm.at[sid], dst_v)

    def body(j, carry):
      pltpu.sync_copy(ones_v, acc.at[dst_v.at[j]], add=True)
      return carry

    lax.fori_loop(0, NCHUNK, body, 0)
    plsc.subcore_barrier()
    pltpu.sync_copy(acc.at[pl.ds(sid * ROWS_PER_TILE, ROWS_PER_TILE)],
                    out_hbm.at[cid, pl.ds(sid * ROWS_PER_TILE, ROWS_PER_TILE)])

  return deg_kernel


def _sc_scatter(fh):
  mesh = plsc.VectorSubcoreMesh(core_axis_name="c", subcore_axis_name="s",
                                num_cores=NC, num_subcores=NS)

  @functools.partial(
      pl.kernel,
      out_type=jax.ShapeDtypeStruct((NC, NPAD, fh), jnp.float32),
      mesh=mesh,
      scratch_types=[
          pltpu.VMEM((IGRP, CHUNK), jnp.int32),
          pltpu.VMEM((IGRP, CHUNK), jnp.int32),
          pltpu.VMEM((NBUF, CHUNK, fh), jnp.float32),
          pltpu.VMEM_SHARED((NNODES, fh), jnp.float32),
          pltpu.VMEM_SHARED((NPAD, fh), jnp.float32),
      ] + [pltpu.SemaphoreType.DMA] * (2 * NBUF),
      compiler_params=pltpu.CompilerParams(use_tc_tiling_on_sc=False),
  )
  def scat_kernel(y_hbm, srcp_hbm, dstp_hbm, zeros_hbm, out_hbm,
                  src_v, dst_v, rows_v, y_spm, acc, *sems):
    cid = lax.axis_index("c")
    sid = lax.axis_index("s")
    gsem = sems[:NBUF]
    ssem = sems[NBUF:]
    pltpu.sync_copy(zeros_hbm, acc.at[pl.ds(sid * ROWS_PER_TILE, ROWS_PER_TILE)])
    pltpu.sync_copy(y_hbm.at[pl.ds(sid * STAGE_PER_TILE, STAGE_PER_TILE),
                             pl.ds(cid * fh, fh)],
                    y_spm.at[pl.ds(sid * STAGE_PER_TILE, STAGE_PER_TILE)])
    plsc.subcore_barrier()

    def group(g, carry):
      pltpu.sync_copy(srcp_hbm.at[sid, pl.ds(g * IGRP, IGRP)], src_v)
      pltpu.sync_copy(dstp_hbm.at[sid, pl.ds(g * IGRP, IGRP)], dst_v)
      for b in range(NBUF - 1):
        pltpu.async_copy(y_spm.at[src_v.at[b]], rows_v.at[b], gsem[b])

      def body(j4, c2):
        for b in range(NBUF):
          j = j4 * NBUF + b
          bb = (b + NBUF - 1) % NBUF
          pltpu.make_async_copy(y_spm.at[src_v.at[j]], rows_v.at[b],
                                gsem[b]).wait()
          pltpu.async_copy(rows_v.at[b], acc.at[dst_v.at[j]], ssem[b],
                           add=True)

          @pl.when(j > 0)
          def _drain_prev():
            pltpu.make_async_copy(rows_v.at[bb], acc.at[dst_v.at[j - 1]],
                                  ssem[bb]).wait()

          @pl.when(j + NBUF - 1 < IGRP)
          def _fire_next():
            pltpu.async_copy(y_spm.at[src_v.at[j + NBUF - 1]], rows_v.at[bb],
                             gsem[bb])
        return c2

      lax.fori_loop(0, IGRP // NBUF, body, 0)
      pltpu.make_async_copy(rows_v.at[(IGRP - 1) % NBUF],
                            acc.at[dst_v.at[IGRP - 1]],
                            ssem[(IGRP - 1) % NBUF]).wait()
      return carry

    lax.fori_loop(0, NCHUNK // IGRP, group, 0)
    plsc.subcore_barrier()
    pltpu.sync_copy(acc.at[pl.ds(sid * ROWS_PER_TILE, ROWS_PER_TILE)],
                    out_hbm.at[cid, pl.ds(sid * ROWS_PER_TILE, ROWS_PER_TILE)])

  return scat_kernel


_RB = 2000


def _mm(x, w):
  n, d = x.shape
  h = w.shape[1]

  def body(x_ref, w_ref, y_ref):
    y_ref[...] = jnp.dot(x_ref[...], w_ref[...],
                         preferred_element_type=jnp.float32)

  return pl.pallas_call(
      body,
      grid=(n // _RB,),
      in_specs=[
          pl.BlockSpec((_RB, d), lambda i: (i, 0)),
          pl.BlockSpec((d, h), lambda i: (0, 0)),
      ],
      out_specs=pl.BlockSpec((_RB, h), lambda i: (i, 0)),
      out_shape=jax.ShapeDtypeStruct((n, h), jnp.float32),
  )(x, w)


def _scale_deg(xw, deg2d):
  n, h = xw.shape

  def body(xw_ref, deg_ref, y_ref, di_ref):
    di = lax.rsqrt(deg_ref[...] + 1.0)
    di_ref[...] = di
    y_ref[...] = xw_ref[...] * di

  return pl.pallas_call(
      body,
      grid=(n // _RB,),
      in_specs=[
          pl.BlockSpec((_RB, h), lambda i: (i, 0)),
          pl.BlockSpec((_RB, 1), lambda i: (i, 0)),
      ],
      out_specs=[
          pl.BlockSpec((_RB, h), lambda i: (i, 0)),
          pl.BlockSpec((_RB, 1), lambda i: (i, 0)),
      ],
      out_shape=[
          jax.ShapeDtypeStruct((n, h), jnp.float32),
          jax.ShapeDtypeStruct((n, 1), jnp.float32),
      ],
  )(xw, deg2d)


def _layer_tail(parts, y, b2d, dinv2, w):
  n, h = y.shape
  fh = h // NC
  h2 = w.shape[1]
  nb = n // _RB

  def body(p_ref, y_ref, b_ref, di_ref, w_ref, yn_ref, v_scr, s_scr, q_scr):
    ph = pl.program_id(0)
    i = pl.program_id(1)
    di = di_ref[...]

    @pl.when(ph == 0)
    def _stats_phase():
      p = jnp.concatenate([p_ref[0], p_ref[1]], axis=1)
      v = di * (p + y_ref[...]) + b_ref[...]
      v_scr[pl.ds(i * _RB, _RB), :] = v

      @pl.when(i == 0)
      def _init():
        s_scr[...] = jnp.zeros_like(s_scr)
        q_scr[...] = jnp.zeros_like(q_scr)

      s_scr[...] += v.sum(axis=0, keepdims=True)
      q_scr[...] += (v * v).sum(axis=0, keepdims=True)

    @pl.when(ph == 1)
    def _mm_phase():
      mu = s_scr[...] / n
      var = (jnp.sum(q_scr[...]) - n * jnp.sum(mu * mu)) / n
      sinv = lax.rsqrt(1e-6 + var)
      z = jax.nn.relu((v_scr[pl.ds(i * _RB, _RB), :] - mu) * sinv)
      yn_ref[...] = jnp.dot(z, w_ref[...],
                            preferred_element_type=jnp.float32) * di

  last = nb - 1
  return pl.pallas_call(
      body,
      grid=(2, nb),
      in_specs=[
          pl.BlockSpec((NC, _RB, fh), lambda ph, i: (0, jnp.where(ph == 0, i, last), 0)),
          pl.BlockSpec((_RB, h), lambda ph, i: (jnp.where(ph == 0, i, last), 0)),
          pl.BlockSpec((1, h), lambda ph, i: (0, 0)),
          pl.BlockSpec((_RB, 1), lambda ph, i: (i, 0)),
          pl.BlockSpec((h, h2), lambda ph, i: (0, 0)),
      ],
      out_specs=pl.BlockSpec((_RB, h2), lambda ph, i: (i, 0)),
      out_shape=jax.ShapeDtypeStruct((n, h2), jnp.float32),
      scratch_shapes=[
          pltpu.VMEM((n, h), jnp.float32),
          pltpu.VMEM((1, h), jnp.float32),
          pltpu.VMEM((1, h), jnp.float32),
      ],
  )(parts, y, b2d, dinv2, w)


def _combine_final(parts, y, b2d, dinv2):
  n, h = y.shape
  fh = h // NC

  def body(p_ref, y_ref, b_ref, di_ref, v_ref):
    p = jnp.concatenate([p_ref[0], p_ref[1]], axis=1)
    v_ref[...] = di_ref[...] * (p + y_ref[...]) + b_ref[...]

  return pl.pallas_call(
      body,
      grid=(n // _RB,),
      in_specs=[
          pl.BlockSpec((NC, _RB, fh), lambda i: (0, i, 0)),
          pl.BlockSpec((_RB, h), lambda i: (i, 0)),
          pl.BlockSpec((1, h), lambda i: (0, 0)),
          pl.BlockSpec((_RB, 1), lambda i: (i, 0)),
      ],
      out_specs=pl.BlockSpec((_RB, h), lambda i: (i, 0)),
      out_shape=jax.ShapeDtypeStruct((n, h), jnp.float32),
  )(parts, y, b2d, dinv2)


def kernel(x, edge_index, W1, b1, W2, b2, W3, b3):
  n = x.shape[0]
  e = edge_index.shape[1]
  pad = EPAD - e
  src = jnp.concatenate([edge_index[0], jnp.zeros((pad,), jnp.int32)])
  dst = jnp.concatenate([edge_index[1], jnp.full((pad,), n, jnp.int32)])
  srcp = src.reshape(NS, NCHUNK, CHUNK)
  dstp = dst.reshape(NS, NCHUNK, CHUNK)

  zeros1d = jnp.zeros((ROWS_PER_TILE,), jnp.float32)
  zeros_h = jnp.zeros((ROWS_PER_TILE, 64), jnp.float32)
  zeros_c = jnp.zeros((ROWS_PER_TILE, 32), jnp.float32)

  degp = _sc_degree()(dstp, zeros1d)
  xw1 = _mm(x, W1)
  y1, dinv2 = _scale_deg(xw1, degp[0].reshape(NPAD, 1))

  scat_h = _sc_scatter(64)
  scat_c = _sc_scatter(32)

  p1 = scat_h(y1, srcp, dstp, zeros_h)
  y2 = _layer_tail(p1, y1, b1.reshape(1, -1), dinv2, W2)

  p2 = scat_h(y2, srcp, dstp, zeros_h)
  y3 = _layer_tail(p2, y2, b2.reshape(1, -1), dinv2, W3)

  p3 = scat_c(y3, srcp, dstp, zeros_c)
  out = _combine_final(p3, y3, b3.reshape(1, -1), dinv2)
  return out


# speedup vs baseline: 1.0512x; 1.0512x over previous
import functools

import jax
import jax.numpy as jnp
from jax import lax
from jax.experimental import pallas as pl
from jax.experimental.pallas import tpu as pltpu
from jax.experimental.pallas import tpu_sc as plsc

NNODES = 10000
NPAD = 10240
NC = 2
NS = 16
CHUNK = 128
NCHUNK = 160
IGRP = 32
NBUF = 4
EPAD = NS * NCHUNK * CHUNK
ROWS_PER_TILE = NPAD // NS
STAGE_PER_TILE = NNODES // NS


def _sc_degree():
  mesh = plsc.VectorSubcoreMesh(core_axis_name="c", subcore_axis_name="s",
                                num_cores=NC, num_subcores=NS)

  @functools.partial(
      pl.kernel,
      out_type=jax.ShapeDtypeStruct((NC, NPAD), jnp.float32),
      mesh=mesh,
      scratch_types=[
          pltpu.VMEM((NCHUNK, CHUNK), jnp.int32),
          pltpu.VMEM((CHUNK,), jnp.float32),
          pltpu.VMEM_SHARED((NPAD,), jnp.float32),
      ],
  )
  def deg_kernel(dstp_hbm, zeros_hbm, out_hbm, dst_v, ones_v, acc):
    cid = lax.axis_index("c")
    sid = lax.axis_index("s")
    for i in range(CHUNK // 16):
      ones_v[pl.ds(i * 16, 16)] = jnp.ones((16,), jnp.float32)
    pltpu.sync_copy(zeros_hbm, acc.at[pl.ds(sid * ROWS_PER_TILE, ROWS_PER_TILE)])
    plsc.subcore_barrier()
    pltpu.sync_copy(dstp_hbm.at[sid], dst_v)

    def body(j, carry):
      pltpu.sync_copy(ones_v, acc.at[dst_v.at[j]], add=True)
      return carry

    lax.fori_loop(0, NCHUNK, body, 0)
    plsc.subcore_barrier()
    pltpu.sync_copy(acc.at[pl.ds(sid * ROWS_PER_TILE, ROWS_PER_TILE)],
                    out_hbm.at[cid, pl.ds(sid * ROWS_PER_TILE, ROWS_PER_TILE)])

  return deg_kernel


def _sc_scatter(fh):
  mesh = plsc.VectorSubcoreMesh(core_axis_name="c", subcore_axis_name="s",
                                num_cores=NC, num_subcores=NS)

  @functools.partial(
      pl.kernel,
      out_type=jax.ShapeDtypeStruct((NC, NPAD, fh), jnp.float32),
      mesh=mesh,
      scratch_types=[
          pltpu.VMEM((IGRP, CHUNK), jnp.int32),
          pltpu.VMEM((IGRP, CHUNK), jnp.int32),
          pltpu.VMEM((NBUF, CHUNK, fh), jnp.float32),
          pltpu.VMEM_SHARED((NNODES, fh), jnp.float32),
          pltpu.VMEM_SHARED((NPAD, fh), jnp.float32),
      ] + [pltpu.SemaphoreType.DMA] * (2 * NBUF),
      compiler_params=pltpu.CompilerParams(use_tc_tiling_on_sc=False),
  )
  def scat_kernel(y_hbm, srcp_hbm, dstp_hbm, zeros_hbm, out_hbm,
                  src_v, dst_v, rows_v, y_spm, acc, *sems):
    cid = lax.axis_index("c")
    sid = lax.axis_index("s")
    gsem = sems[:NBUF]
    ssem = sems[NBUF:]
    pltpu.sync_copy(zeros_hbm, acc.at[pl.ds(sid * ROWS_PER_TILE, ROWS_PER_TILE)])
    pltpu.sync_copy(y_hbm.at[pl.ds(sid * STAGE_PER_TILE, STAGE_PER_TILE),
                             pl.ds(cid * fh, fh)],
                    y_spm.at[pl.ds(sid * STAGE_PER_TILE, STAGE_PER_TILE)])
    plsc.subcore_barrier()

    def group(g, carry):
      pltpu.sync_copy(srcp_hbm.at[sid, pl.ds(g * IGRP, IGRP)], src_v)
      pltpu.sync_copy(dstp_hbm.at[sid, pl.ds(g * IGRP, IGRP)], dst_v)
      for b in range(NBUF - 1):
        pltpu.async_copy(y_spm.at[src_v.at[b]], rows_v.at[b], gsem[b])

      def body(j4, c2):
        for b in range(NBUF):
          j = j4 * NBUF + b
          bb = (b + NBUF - 1) % NBUF
          pltpu.make_async_copy(y_spm.at[src_v.at[j]], rows_v.at[b],
                                gsem[b]).wait()
          pltpu.async_copy(rows_v.at[b], acc.at[dst_v.at[j]], ssem[b],
                           add=True)

          @pl.when(j > 0)
          def _drain_prev():
            pltpu.make_async_copy(rows_v.at[bb], acc.at[dst_v.at[j - 1]],
                                  ssem[bb]).wait()

          @pl.when(j + NBUF - 1 < IGRP)
          def _fire_next():
            pltpu.async_copy(y_spm.at[src_v.at[j + NBUF - 1]], rows_v.at[bb],
                             gsem[bb])
        return c2

      lax.fori_loop(0, IGRP // NBUF, body, 0)
      pltpu.make_async_copy(rows_v.at[(IGRP - 1) % NBUF],
                            acc.at[dst_v.at[IGRP - 1]],
                            ssem[(IGRP - 1) % NBUF]).wait()
      return carry

    lax.fori_loop(0, NCHUNK // IGRP, group, 0)
    plsc.subcore_barrier()
    pltpu.sync_copy(acc.at[pl.ds(sid * ROWS_PER_TILE, ROWS_PER_TILE)],
                    out_hbm.at[cid, pl.ds(sid * ROWS_PER_TILE, ROWS_PER_TILE)])

  return scat_kernel


_RB = 2000


def _mm(x, w):
  n, d = x.shape
  h = w.shape[1]

  def body(x_ref, w_ref, y_ref):
    y_ref[...] = jnp.dot(x_ref[...], w_ref[...],
                         preferred_element_type=jnp.float32)

  return pl.pallas_call(
      body,
      grid=(n // _RB,),
      in_specs=[
          pl.BlockSpec((_RB, d), lambda i: (i, 0)),
          pl.BlockSpec((d, h), lambda i: (0, 0)),
      ],
      out_specs=pl.BlockSpec((_RB, h), lambda i: (i, 0)),
      out_shape=jax.ShapeDtypeStruct((n, h), jnp.float32),
  )(x, w)


def _scale_deg(xw, deg2d):
  n, h = xw.shape

  def body(xw_ref, deg_ref, y_ref, di_ref):
    di = lax.rsqrt(deg_ref[...] + 1.0)
    di_ref[...] = di
    y_ref[...] = xw_ref[...] * di

  return pl.pallas_call(
      body,
      grid=(n // _RB,),
      in_specs=[
          pl.BlockSpec((_RB, h), lambda i: (i, 0)),
          pl.BlockSpec((_RB, 1), lambda i: (i, 0)),
      ],
      out_specs=[
          pl.BlockSpec((_RB, h), lambda i: (i, 0)),
          pl.BlockSpec((_RB, 1), lambda i: (i, 0)),
      ],
      out_shape=[
          jax.ShapeDtypeStruct((n, h), jnp.float32),
          jax.ShapeDtypeStruct((n, 1), jnp.float32),
      ],
  )(xw, deg2d)


def _layer_tail(parts, y, b2d, dinv2, w):
  n, h = y.shape
  fh = h // NC
  h2 = w.shape[1]
  nb = n // _RB

  def body(p_ref, y_ref, b_ref, di_ref, w_ref, yn_ref, v_scr, s_scr, q_scr):
    ph = pl.program_id(0)
    i = pl.program_id(1)
    di = di_ref[...]

    @pl.when(ph == 0)
    def _stats_phase():
      p = jnp.concatenate([p_ref[0], p_ref[1]], axis=1)
      v = di * (p + y_ref[...]) + b_ref[...]
      v_scr[pl.ds(i * _RB, _RB), :] = v

      @pl.when(i == 0)
      def _init():
        s_scr[...] = jnp.zeros_like(s_scr)
        q_scr[...] = jnp.zeros_like(q_scr)

      s_scr[...] += v.sum(axis=0, keepdims=True)
      q_scr[...] += (v * v).sum(axis=0, keepdims=True)

    @pl.when(ph == 1)
    def _mm_phase():
      mu = s_scr[...] / n
      var = (jnp.sum(q_scr[...]) - n * jnp.sum(mu * mu)) / n
      sinv = lax.rsqrt(1e-6 + var)
      z = jax.nn.relu((v_scr[pl.ds(i * _RB, _RB), :] - mu) * sinv)
      yn_ref[...] = jnp.dot(z, w_ref[...],
                            preferred_element_type=jnp.float32) * di

  last = nb - 1
  return pl.pallas_call(
      body,
      grid=(2, nb),
      in_specs=[
          pl.BlockSpec((NC, _RB, fh), lambda ph, i: (0, jnp.where(ph == 0, i, last), 0)),
          pl.BlockSpec((_RB, h), lambda ph, i: (jnp.where(ph == 0, i, last), 0)),
          pl.BlockSpec((1, h), lambda ph, i: (0, 0)),
          pl.BlockSpec((_RB, 1), lambda ph, i: (i, 0)),
          pl.BlockSpec((h, h2), lambda ph, i: (0, 0)),
      ],
      out_specs=pl.BlockSpec((_RB, h2), lambda ph, i: (i, 0)),
      out_shape=jax.ShapeDtypeStruct((n, h2), jnp.float32),
      scratch_shapes=[
          pltpu.VMEM((n, h), jnp.float32),
          pltpu.VMEM((1, h), jnp.float32),
          pltpu.VMEM((1, h), jnp.float32),
      ],
  )(parts, y, b2d, dinv2, w)


def _combine_final(parts, y, b2d, dinv2):
  n, h = y.shape
  fh = h // NC

  def body(p_ref, y_ref, b_ref, di_ref, v_ref):
    p = jnp.concatenate([p_ref[0], p_ref[1]], axis=1)
    v_ref[...] = di_ref[...] * (p + y_ref[...]) + b_ref[...]

  return pl.pallas_call(
      body,
      grid=(n // _RB,),
      in_specs=[
          pl.BlockSpec((NC, _RB, fh), lambda i: (0, i, 0)),
          pl.BlockSpec((_RB, h), lambda i: (i, 0)),
          pl.BlockSpec((1, h), lambda i: (0, 0)),
          pl.BlockSpec((_RB, 1), lambda i: (i, 0)),
      ],
      out_specs=pl.BlockSpec((_RB, h), lambda i: (i, 0)),
      out_shape=jax.ShapeDtypeStruct((n, h), jnp.float32),
  )(parts, y, b2d, dinv2)


def kernel(x, edge_index, W1, b1, W2, b2, W3, b3):
  n = x.shape[0]
  e = edge_index.shape[1]
  pad = EPAD - e
  src = jnp.concatenate([edge_index[0], jnp.zeros((pad,), jnp.int32)])
  dst = jnp.concatenate([edge_index[1], jnp.full((pad,), n, jnp.int32)])
  srcp = src.reshape(NS, NCHUNK, CHUNK)
  dstp = dst.reshape(NS, NCHUNK, CHUNK)

  zeros1d = jnp.zeros((ROWS_PER_TILE,), jnp.float32)
  zeros_h = jnp.zeros((ROWS_PER_TILE, 64), jnp.float32)
  zeros_c = jnp.zeros((ROWS_PER_TILE, 32), jnp.float32)

  degp = _sc_degree()(dstp, zeros1d)
  xw1 = _mm(x, W1)
  y1, dinv2 = _scale_deg(xw1, degp[0].reshape(NPAD, 1))

  scat_h = _sc_scatter(64)
  scat_c = _sc_scatter(32)

  p1 = scat_h(y1, srcp, dstp, zeros_h)
  y2 = _layer_tail(p1, y1, b1.reshape(1, -1), dinv2, W2)

  p2 = scat_h(y2, srcp, dstp, zeros_h)
  y3 = _layer_tail(p2, y2, b2.reshape(1, -1), dinv2, W3)

  p3 = scat_c(y3, srcp, dstp, zeros_c)
  out = _combine_final(p3, y3, b3.reshape(1, -1), dinv2)
  return out


# speedup vs baseline: 1.0656x; 1.0136x over previous
import functools

import jax
import jax.numpy as jnp
from jax import lax
from jax.experimental import pallas as pl
from jax.experimental.pallas import tpu as pltpu
from jax.experimental.pallas import tpu_sc as plsc

NNODES = 10000
NPAD = 10240
NC = 2
NS = 16
CHUNK = 128
NCHUNK = 160
IGRP = 32
NBUF = 4
EPAD = NS * NCHUNK * CHUNK
ROWS_PER_TILE = NPAD // NS
STAGE_PER_TILE = NNODES // NS


def _sc_degree():
  mesh = plsc.VectorSubcoreMesh(core_axis_name="c", subcore_axis_name="s",
                                num_cores=NC, num_subcores=NS)

  @functools.partial(
      pl.kernel,
      out_type=jax.ShapeDtypeStruct((NC, NPAD), jnp.float32),
      mesh=mesh,
      scratch_types=[
          pltpu.VMEM((NCHUNK // NC, CHUNK), jnp.int32),
          pltpu.VMEM((CHUNK,), jnp.float32),
          pltpu.VMEM_SHARED((NPAD,), jnp.float32),
      ],
  )
  def deg_kernel(dstp_hbm, zeros_hbm, out_hbm, dst_v, ones_v, acc):
    cid = lax.axis_index("c")
    sid = lax.axis_index("s")
    for i in range(CHUNK // 16):
      ones_v[pl.ds(i * 16, 16)] = jnp.ones((16,), jnp.float32)
    pltpu.sync_copy(zeros_hbm, acc.at[pl.ds(sid * ROWS_PER_TILE, ROWS_PER_TILE)])
    plsc.subcore_barrier()
    pltpu.sync_copy(dstp_hbm.at[sid, pl.ds(cid * (NCHUNK // NC), NCHUNK // NC)],
                    dst_v)

    def body(j, carry):
      pltpu.sync_copy(ones_v, acc.at[dst_v.at[j]], add=True)
      return carry

    lax.fori_loop(0, NCHUNK // NC, body, 0)
    plsc.subcore_barrier()
    pltpu.sync_copy(acc.at[pl.ds(sid * ROWS_PER_TILE, ROWS_PER_TILE)],
                    out_hbm.at[cid, pl.ds(sid * ROWS_PER_TILE, ROWS_PER_TILE)])

  return deg_kernel


def _sc_scatter(fh):
  mesh = plsc.VectorSubcoreMesh(core_axis_name="c", subcore_axis_name="s",
                                num_cores=NC, num_subcores=NS)

  @functools.partial(
      pl.kernel,
      out_type=jax.ShapeDtypeStruct((NC, NPAD, fh), jnp.float32),
      mesh=mesh,
      scratch_types=[
          pltpu.VMEM((IGRP, CHUNK), jnp.int32),
          pltpu.VMEM((IGRP, CHUNK), jnp.int32),
          pltpu.VMEM((NBUF, CHUNK, fh), jnp.float32),
          pltpu.VMEM_SHARED((NNODES, fh), jnp.float32),
          pltpu.VMEM_SHARED((NPAD, fh), jnp.float32),
      ] + [pltpu.SemaphoreType.DMA] * (2 * NBUF),
      compiler_params=pltpu.CompilerParams(use_tc_tiling_on_sc=False),
  )
  def scat_kernel(y_hbm, srcp_hbm, dstp_hbm, zeros_hbm, out_hbm,
                  src_v, dst_v, rows_v, y_spm, acc, *sems):
    cid = lax.axis_index("c")
    sid = lax.axis_index("s")
    gsem = sems[:NBUF]
    ssem = sems[NBUF:]
    pltpu.sync_copy(zeros_hbm, acc.at[pl.ds(sid * ROWS_PER_TILE, ROWS_PER_TILE)])
    pltpu.sync_copy(y_hbm.at[pl.ds(sid * STAGE_PER_TILE, STAGE_PER_TILE),
                             pl.ds(cid * fh, fh)],
                    y_spm.at[pl.ds(sid * STAGE_PER_TILE, STAGE_PER_TILE)])
    plsc.subcore_barrier()

    def group(g, carry):
      pltpu.sync_copy(srcp_hbm.at[sid, pl.ds(g * IGRP, IGRP)], src_v)
      pltpu.sync_copy(dstp_hbm.at[sid, pl.ds(g * IGRP, IGRP)], dst_v)
      for b in range(NBUF - 1):
        pltpu.async_copy(y_spm.at[src_v.at[b]], rows_v.at[b], gsem[b])

      def body(j4, c2):
        for b in range(NBUF):
          j = j4 * NBUF + b
          bb = (b + NBUF - 1) % NBUF
          pltpu.make_async_copy(y_spm.at[src_v.at[j]], rows_v.at[b],
                                gsem[b]).wait()
          pltpu.async_copy(rows_v.at[b], acc.at[dst_v.at[j]], ssem[b],
                           add=True)

          @pl.when(j > 0)
          def _drain_prev():
            pltpu.make_async_copy(rows_v.at[bb], acc.at[dst_v.at[j - 1]],
                                  ssem[bb]).wait()

          @pl.when(j + NBUF - 1 < IGRP)
          def _fire_next():
            pltpu.async_copy(y_spm.at[src_v.at[j + NBUF - 1]], rows_v.at[bb],
                             gsem[bb])
        return c2

      lax.fori_loop(0, IGRP // NBUF, body, 0)
      pltpu.make_async_copy(rows_v.at[(IGRP - 1) % NBUF],
                            acc.at[dst_v.at[IGRP - 1]],
                            ssem[(IGRP - 1) % NBUF]).wait()
      return carry

    lax.fori_loop(0, NCHUNK // IGRP, group, 0)
    plsc.subcore_barrier()
    pltpu.sync_copy(acc.at[pl.ds(sid * ROWS_PER_TILE, ROWS_PER_TILE)],
                    out_hbm.at[cid, pl.ds(sid * ROWS_PER_TILE, ROWS_PER_TILE)])

  return scat_kernel


_RB = 2000


def _mm(x, w):
  n, d = x.shape
  h = w.shape[1]

  def body(x_ref, w_ref, y_ref):
    y_ref[...] = jnp.dot(x_ref[...], w_ref[...],
                         preferred_element_type=jnp.float32)

  return pl.pallas_call(
      body,
      grid=(n // _RB,),
      in_specs=[
          pl.BlockSpec((_RB, d), lambda i: (i, 0)),
          pl.BlockSpec((d, h), lambda i: (0, 0)),
      ],
      out_specs=pl.BlockSpec((_RB, h), lambda i: (i, 0)),
      out_shape=jax.ShapeDtypeStruct((n, h), jnp.float32),
  )(x, w)


def _scale_deg(xw, degp3):
  n, h = xw.shape

  def body(xw_ref, deg_ref, y_ref, di_ref):
    di = lax.rsqrt(deg_ref[0] + deg_ref[1] + 1.0)
    di_ref[...] = di
    y_ref[...] = xw_ref[...] * di

  return pl.pallas_call(
      body,
      grid=(n // _RB,),
      in_specs=[
          pl.BlockSpec((_RB, h), lambda i: (i, 0)),
          pl.BlockSpec((NC, _RB, 1), lambda i: (0, i, 0)),
      ],
      out_specs=[
          pl.BlockSpec((_RB, h), lambda i: (i, 0)),
          pl.BlockSpec((_RB, 1), lambda i: (i, 0)),
      ],
      out_shape=[
          jax.ShapeDtypeStruct((n, h), jnp.float32),
          jax.ShapeDtypeStruct((n, 1), jnp.float32),
      ],
  )(xw, degp3)


def _layer_tail(parts, y, b2d, dinv2, w):
  n, h = y.shape
  fh = h // NC
  h2 = w.shape[1]
  nb = n // _RB

  def body(p_ref, y_ref, b_ref, di_ref, w_ref, yn_ref, v_scr, s_scr, q_scr):
    ph = pl.program_id(0)
    i = pl.program_id(1)
    di = di_ref[...]

    @pl.when(ph == 0)
    def _stats_phase():
      p = jnp.concatenate([p_ref[0], p_ref[1]], axis=1)
      v = di * (p + y_ref[...]) + b_ref[...]
      v_scr[pl.ds(i * _RB, _RB), :] = v

      @pl.when(i == 0)
      def _init():
        s_scr[...] = jnp.zeros_like(s_scr)
        q_scr[...] = jnp.zeros_like(q_scr)

      s_scr[...] += v.sum(axis=0, keepdims=True)
      q_scr[...] += (v * v).sum(axis=0, keepdims=True)

    @pl.when(ph == 1)
    def _mm_phase():
      mu = s_scr[...] / n
      var = (jnp.sum(q_scr[...]) - n * jnp.sum(mu * mu)) / n
      sinv = lax.rsqrt(1e-6 + var)
      z = jax.nn.relu((v_scr[pl.ds(i * _RB, _RB), :] - mu) * sinv)
      yn_ref[...] = jnp.dot(z, w_ref[...],
                            preferred_element_type=jnp.float32) * di

  last = nb - 1
  return pl.pallas_call(
      body,
      grid=(2, nb),
      in_specs=[
          pl.BlockSpec((NC, _RB, fh), lambda ph, i: (0, jnp.where(ph == 0, i, last), 0)),
          pl.BlockSpec((_RB, h), lambda ph, i: (jnp.where(ph == 0, i, last), 0)),
          pl.BlockSpec((1, h), lambda ph, i: (0, 0)),
          pl.BlockSpec((_RB, 1), lambda ph, i: (i, 0)),
          pl.BlockSpec((h, h2), lambda ph, i: (0, 0)),
      ],
      out_specs=pl.BlockSpec((_RB, h2), lambda ph, i: (i, 0)),
      out_shape=jax.ShapeDtypeStruct((n, h2), jnp.float32),
      scratch_shapes=[
          pltpu.VMEM((n, h), jnp.float32),
          pltpu.VMEM((1, h), jnp.float32),
          pltpu.VMEM((1, h), jnp.float32),
      ],
  )(parts, y, b2d, dinv2, w)


def _combine_final(parts, y, b2d, dinv2):
  n, h = y.shape
  fh = h // NC

  def body(p_ref, y_ref, b_ref, di_ref, v_ref):
    p = jnp.concatenate([p_ref[0], p_ref[1]], axis=1)
    v_ref[...] = di_ref[...] * (p + y_ref[...]) + b_ref[...]

  return pl.pallas_call(
      body,
      grid=(n // _RB,),
      in_specs=[
          pl.BlockSpec((NC, _RB, fh), lambda i: (0, i, 0)),
          pl.BlockSpec((_RB, h), lambda i: (i, 0)),
          pl.BlockSpec((1, h), lambda i: (0, 0)),
          pl.BlockSpec((_RB, 1), lambda i: (i, 0)),
      ],
      out_specs=pl.BlockSpec((_RB, h), lambda i: (i, 0)),
      out_shape=jax.ShapeDtypeStruct((n, h), jnp.float32),
  )(parts, y, b2d, dinv2)


def kernel(x, edge_index, W1, b1, W2, b2, W3, b3):
  n = x.shape[0]
  e = edge_index.shape[1]
  pad = EPAD - e
  src = jnp.concatenate([edge_index[0], jnp.zeros((pad,), jnp.int32)])
  dst = jnp.concatenate([edge_index[1], jnp.full((pad,), n, jnp.int32)])
  srcp = src.reshape(NS, NCHUNK, CHUNK)
  dstp = dst.reshape(NS, NCHUNK, CHUNK)

  zeros1d = jnp.zeros((ROWS_PER_TILE,), jnp.float32)
  zeros_h = jnp.zeros((ROWS_PER_TILE, 64), jnp.float32)
  zeros_c = jnp.zeros((ROWS_PER_TILE, 32), jnp.float32)

  degp = _sc_degree()(dstp, zeros1d)
  xw1 = _mm(x, W1)
  y1, dinv2 = _scale_deg(xw1, degp.reshape(NC, NPAD, 1))

  scat_h = _sc_scatter(64)
  scat_c = _sc_scatter(32)

  p1 = scat_h(y1, srcp, dstp, zeros_h)
  y2 = _layer_tail(p1, y1, b1.reshape(1, -1), dinv2, W2)

  p2 = scat_h(y2, srcp, dstp, zeros_h)
  y3 = _layer_tail(p2, y2, b2.reshape(1, -1), dinv2, W3)

  p3 = scat_c(y3, srcp, dstp, zeros_c)
  out = _combine_final(p3, y3, b3.reshape(1, -1), dinv2)
  return out
